# initial kernel scaffold (unmeasured)
import jax
import jax.numpy as jnp
from jax import lax
from jax.experimental import pallas as pl
from jax.experimental.pallas import tpu as pltpu


def kernel(
    x,
):
    def body(*refs):
        pass

    out_shape = jax.ShapeDtypeStruct(..., jnp.float32)
    return pl.pallas_call(body, out_shape=out_shape)(...)


# baseline (device time: 14911 ns/iter reference)
import jax
import jax.numpy as jnp
from jax import lax
from jax.experimental import pallas as pl
from jax.experimental.pallas import tpu as pltpu

N_Y = 4


def kernel(x):
    m_per, n = x.shape

    def body(x_ref, out_ref, comm_ref, send_sems, recv_sems):
        my_x = lax.axis_index("x")
        my_y = lax.axis_index("y")
        my_z = lax.axis_index("z")
        left = (my_y - 1) % N_Y
        right = (my_y + 1) % N_Y

        barrier_sem = pltpu.get_barrier_semaphore()
        for nbr in [left, right]:
            pl.semaphore_signal(
                barrier_sem, inc=1,
                device_id=(my_x, nbr, my_z),
                device_id_type=pl.DeviceIdType.MESH,
            )
        pl.semaphore_wait(barrier_sem, 2)

        mine = x_ref[:, :].astype(jnp.bfloat16)
        out_ref[pl.ds(my_y * m_per, m_per), :] = mine
        comm_ref[0] = mine

        for h in range(N_Y - 1):
            send_slot = h % 2
            recv_slot = (h + 1) % 2
            rdma = pltpu.make_async_remote_copy(
                src_ref=comm_ref.at[send_slot],
                dst_ref=comm_ref.at[recv_slot],
                send_sem=send_sems.at[send_slot],
                recv_sem=recv_sems.at[recv_slot],
                device_id=(my_x, right, my_z),
                device_id_type=pl.DeviceIdType.MESH,
            )
            rdma.start()
            rdma.wait()

            origin = (my_y - h - 1) % N_Y
            out_ref[pl.ds(origin * m_per, m_per), :] = comm_ref[recv_slot]

    return pl.pallas_call(
        body,
        out_shape=jax.ShapeDtypeStruct((N_Y * m_per, n), jnp.bfloat16),
        in_specs=[pl.BlockSpec(memory_space=pltpu.VMEM)],
        out_specs=pl.BlockSpec(memory_space=pltpu.VMEM),
        scratch_shapes=[
            pltpu.VMEM((2, m_per, n), jnp.bfloat16),
            pltpu.SemaphoreType.DMA((2,)),
            pltpu.SemaphoreType.DMA((2,)),
        ],
        compiler_params=pltpu.CompilerParams(collective_id=0),
    )(x)


# device time: 11653 ns/iter; 1.2796x vs baseline; 1.2796x over previous
import jax
import jax.numpy as jnp
from jax import lax
from jax.experimental import pallas as pl
from jax.experimental.pallas import tpu as pltpu

N_Y = 4


def kernel(x):
    m_per, n = x.shape

    def body(x_ref, out_ref, send_sems, recv_sems):
        my_x = lax.axis_index("x")
        my_y = lax.axis_index("y")
        my_z = lax.axis_index("z")

        def dev(y):
            return (my_x, jnp.clip(y, 0, N_Y - 1), my_z)

        def chunk(ref, c):
            return ref.at[pl.ds(c * m_per, m_per), :]

        barrier_sem = pltpu.get_barrier_semaphore()
        for off in (-2, -1, 1, 2):
            tgt = my_y + off
            @pl.when((tgt >= 0) & (tgt < N_Y))
            def _():
                pl.semaphore_signal(
                    barrier_sem, inc=1,
                    device_id=dev(tgt),
                    device_id_type=pl.DeviceIdType.MESH,
                )
        n_partners = 2 + jnp.where((my_y >= 1) & (my_y <= 2), 1, 0)
        @pl.when(n_partners == 2)
        def _():
            pl.semaphore_wait(barrier_sem, 2)
        @pl.when(n_partners == 3)
        def _():
            pl.semaphore_wait(barrier_sem, 3)

        chunk(out_ref, my_y)[...] = x_ref[:, :].astype(jnp.bfloat16)

        started = []

        for slot, off in enumerate((-2, -1, 1, 2)):
            tgt = my_y + off
            rdma = pltpu.make_async_remote_copy(
                src_ref=chunk(out_ref, my_y),
                dst_ref=chunk(out_ref, my_y),
                send_sem=send_sems.at[slot],
                recv_sem=recv_sems.at[my_y],
                device_id=dev(tgt),
                device_id_type=pl.DeviceIdType.MESH,
            )
            pred = (tgt >= 0) & (tgt < N_Y)
            @pl.when(pred)
            def _(rdma=rdma):
                rdma.start()
            started.append((rdma, pred))

        def recv_desc(c):
            return pltpu.make_async_remote_copy(
                src_ref=chunk(out_ref, c),
                dst_ref=chunk(out_ref, c),
                send_sem=send_sems.at[4],
                recv_sem=recv_sems.at[c],
                device_id=dev(my_y),
                device_id_type=pl.DeviceIdType.MESH,
            )

        fwd_chunk = jnp.where(my_y == 1, N_Y - 1, 0)
        fwd_tgt = jnp.where(my_y == 1, 0, N_Y - 1)
        fwd_rdma = pltpu.make_async_remote_copy(
            src_ref=chunk(out_ref, fwd_chunk),
            dst_ref=chunk(out_ref, fwd_chunk),
            send_sem=send_sems.at[4],
            recv_sem=recv_sems.at[fwd_chunk],
            device_id=dev(fwd_tgt),
            device_id_type=pl.DeviceIdType.MESH,
        )
        fwd_pred = (my_y == 1) | (my_y == 2)
        @pl.when(fwd_pred)
        def _():
            recv_desc(fwd_chunk).wait_recv()
            fwd_rdma.start()
        started.append((fwd_rdma, fwd_pred))

        for c in range(N_Y):
            @pl.when((c != my_y) & ~(fwd_pred & (c == fwd_chunk)))
            def _(c=c):
                recv_desc(c).wait_recv()

        for rdma, pred in started:
            @pl.when(pred)
            def _(rdma=rdma):
                rdma.wait_send()

    return pl.pallas_call(
        body,
        out_shape=jax.ShapeDtypeStruct((N_Y * m_per, n), jnp.bfloat16),
        in_specs=[pl.BlockSpec(memory_space=pltpu.VMEM)],
        out_specs=pl.BlockSpec(memory_space=pltpu.VMEM),
        scratch_shapes=[
            pltpu.SemaphoreType.DMA((5,)),
            pltpu.SemaphoreType.DMA((N_Y,)),
        ],
        compiler_params=pltpu.CompilerParams(collective_id=0),
    )(x)
